# baseline (speedup 1.0000x reference)
import functools

import jax
import jax.numpy as jnp
from jax import lax
from jax.experimental import pallas as pl
from jax.experimental.pallas import tpu as pltpu
from jax.experimental.pallas import tpu_sc as plsc

L = 20
BATCH = 16384

_NC = 2
_NS = 16
_NW = _NC * _NS
_ROWS = BATCH // _NW
_GROUPS = _ROWS // 16
_GCHUNK = 128


def _ewf_body(x_hbm, aux_hbm, out_hbm, x_v, idx_v, rows_v, sem):
    wid = lax.axis_index("s") * _NC + lax.axis_index("c")
    base = wid * _ROWS

    pltpu.sync_copy(x_hbm.at[pl.ds(base * L, _ROWS * L)], x_v)

    lane20 = lax.iota(jnp.int32, 16) * L
    half = jnp.full((16,), (2.0 ** L - 1.0) / 2.0, dtype=jnp.float32)

    def group(g, carry):
        row_base = g * (16 * L) + lane20
        acc = half
        for i in range(L):
            v = plsc.load_gather(x_v, [row_base + i])
            acc = acc + v * (2.0 ** (L - 2 - i))
        idx_v[pl.ds(g * 16, 16)] = acc.astype(jnp.int32)
        return carry

    lax.fori_loop(0, _GROUPS, group, 0)

    for c in range(_ROWS // _GCHUNK):
        pltpu.async_copy(
            aux_hbm.at[idx_v.at[pl.ds(c * _GCHUNK, _GCHUNK)]],
            rows_v.at[pl.ds(c * _GCHUNK, _GCHUNK)],
            sem,
        ).wait()

    pltpu.sync_copy(rows_v, out_hbm.at[pl.ds(base, _ROWS)])


@jax.jit
def _ewf(x, aux):
    mesh = plsc.VectorSubcoreMesh(
        core_axis_name="c", subcore_axis_name="s",
        num_cores=_NC, num_subcores=_NS,
    )
    return pl.kernel(
        _ewf_body,
        out_type=jax.ShapeDtypeStruct((BATCH,), jnp.float32),
        mesh=mesh,
        scratch_types=[
            pltpu.VMEM((_ROWS * L,), jnp.float32),
            pltpu.VMEM((_ROWS,), jnp.int32),
            pltpu.VMEM((_ROWS,), jnp.float32),
            pltpu.SemaphoreType.DMA,
        ],
        compiler_params=pltpu.CompilerParams(needs_layout_passes=False),
    )(x.reshape(-1), aux)


def kernel(x, aux, j1):
    del j1
    return _ewf(x, aux)


# speedup vs baseline: 1.2579x; 1.2579x over previous
import jax
import jax.numpy as jnp
from jax import lax
from jax.experimental import pallas as pl
from jax.experimental.pallas import tpu as pltpu
from jax.experimental.pallas import tpu_sc as plsc

L = 20
BATCH = 16384

_NC = 2
_NS = 16
_NW = _NC * _NS
_ROWS = BATCH // _NW
_CHUNK = 128
_NCHUNK = _ROWS // _CHUNK
_GROUPS = _CHUNK // 16


def _ewf_body(x_hbm, aux_hbm, out_hbm, x_v, idx_v, rows_v, xsem, gsem):
    wid = lax.axis_index("s") * _NC + lax.axis_index("c")
    base = wid * _ROWS

    xcps = [
        pltpu.async_copy(
            x_hbm.at[pl.ds(base + c * _CHUNK, _CHUNK)],
            x_v.at[pl.ds(c * _CHUNK, _CHUNK)],
            xsem,
        )
        for c in range(_NCHUNK)
    ]

    lane = lax.iota(jnp.int32, 16)
    half = jnp.full((16,), (2.0 ** L - 1.0) / 2.0, dtype=jnp.float32)

    gcps = []
    for c in range(_NCHUNK):
        xcps[c].wait()

        def group(g, carry, c=c):
            row = c * _CHUNK + g * 16 + lane
            acc = half
            for i in range(L):
                col = jnp.full((16,), i, dtype=jnp.int32)
                v = plsc.load_gather(x_v, [row, col])
                acc = acc + v * (2.0 ** (L - 2 - i))
            idx_v[pl.ds(c * _CHUNK + g * 16, 16)] = acc.astype(jnp.int32)
            return carry

        lax.fori_loop(0, _GROUPS, group, 0)

        gcps.append(
            pltpu.async_copy(
                aux_hbm.at[idx_v.at[pl.ds(c * _CHUNK, _CHUNK)]],
                rows_v.at[pl.ds(c * _CHUNK, _CHUNK)],
                gsem,
            )
        )

    for cp in gcps:
        cp.wait()

    pltpu.sync_copy(rows_v, out_hbm.at[pl.ds(base, _ROWS)])


@jax.jit
def _ewf(x, aux):
    mesh = plsc.VectorSubcoreMesh(
        core_axis_name="c", subcore_axis_name="s",
        num_cores=_NC, num_subcores=_NS,
    )
    return pl.kernel(
        _ewf_body,
        out_type=jax.ShapeDtypeStruct((BATCH,), jnp.float32),
        mesh=mesh,
        scratch_types=[
            pltpu.VMEM((_ROWS, L), jnp.float32),
            pltpu.VMEM((_ROWS,), jnp.int32),
            pltpu.VMEM((_ROWS,), jnp.float32),
            pltpu.SemaphoreType.DMA,
            pltpu.SemaphoreType.DMA,
        ],
        compiler_params=pltpu.CompilerParams(needs_layout_passes=False),
    )(x, aux)


def kernel(x, aux, j1):
    del j1
    return _ewf(x, aux)
